# Initial kernel scaffold; baseline (speedup 1.0000x reference)
import jax
import jax.numpy as jnp
from jax.experimental import pallas as pl


def kernel(x, params, edge_index, batch, v1_local, v2_local, seat):
    raise NotImplementedError("write your pallas kernel here")


# speedup vs baseline: 58.0690x; 58.0690x over previous
import functools

import jax
import jax.numpy as jnp
from jax import lax
from jax.experimental import pallas as pl
from jax.experimental.pallas import tpu as pltpu
from jax.experimental.pallas import tpu_sc as plsc

N = 27648
E = 442368
DIM = 128
HEADS = 4
HID = 32
HALF = 64
B = 512
NPG = 54
MLP_IN = DIM * 3 + 8

NTILES = 16
ROWS_T = N // NTILES
EDG_T = E // NTILES
CHUNK = 96
NCHUNK = EDG_T // CHUNK
PACK = 72

ROW_BLK = 512
NBLK = N // ROW_BLK


def _inproj_body(x_ref, w_ref, b_ref, o_ref):
    h = jnp.dot(x_ref[...], w_ref[...], preferred_element_type=jnp.float32)
    o_ref[...] = jnp.maximum(h + b_ref[...], 0.0)


def _inproj(x, w, b):
    return pl.pallas_call(
        _inproj_body,
        grid=(NBLK,),
        in_specs=[
            pl.BlockSpec((ROW_BLK, DIM), lambda i: (i, 0)),
            pl.BlockSpec((DIM, DIM), lambda i: (0, 0)),
            pl.BlockSpec((1, DIM), lambda i: (0, 0)),
        ],
        out_specs=pl.BlockSpec((ROW_BLK, DIM), lambda i: (i, 0)),
        out_shape=jax.ShapeDtypeStruct((N, DIM), jnp.float32),
    )(x, w, b.reshape(1, DIM))


def _pre_body(h_ref, w_ref, ms_ref, md_ref,
              hh2_ref, ad2_ref, st_ref):
    p = pl.program_id(0)
    i = pl.program_id(1)
    hh = jnp.dot(h_ref[...], w_ref[...], preferred_element_type=jnp.float32)
    a_s = jnp.dot(hh, ms_ref[...], preferred_element_type=jnp.float32)
    a_d = jnp.dot(hh, md_ref[...], preferred_element_type=jnp.float32)
    pad = jnp.zeros((ROW_BLK, PACK - HALF - 2), jnp.float32)
    half = jnp.where(p == 0, hh[:, :HALF], hh[:, HALF:])
    spair = jnp.where(p == 0, a_s[:, 0:2], a_s[:, 2:4])
    hh2_ref[...] = jnp.concatenate([half, spair, pad], axis=1)
    ad2_ref[...] = jnp.where(p == 0, a_d[:, 0:2], a_d[:, 2:4])
    row = jnp.concatenate(
        [jnp.max(a_s, axis=0), jnp.max(a_d, axis=0),
         jnp.full((DIM - 2 * HEADS,), -1e30, jnp.float32)])
    blk = jnp.broadcast_to(row[None, :], (8, DIM))
    first = jnp.logical_and(p == 0, i == 0)
    prev = jnp.where(first, jnp.full((8, DIM), -1e30, jnp.float32), st_ref[...])
    st_ref[...] = jnp.maximum(prev, blk)


def _pre(h, w, asrc_mat, adst_mat):
    return pl.pallas_call(
        _pre_body,
        grid=(2, NBLK),
        in_specs=[
            pl.BlockSpec((ROW_BLK, DIM), lambda p, i: (i, 0)),
            pl.BlockSpec((DIM, DIM), lambda p, i: (0, 0)),
            pl.BlockSpec((DIM, HEADS), lambda p, i: (0, 0)),
            pl.BlockSpec((DIM, HEADS), lambda p, i: (0, 0)),
        ],
        out_specs=[
            pl.BlockSpec((ROW_BLK, PACK), lambda p, i: (p * NBLK + i, 0)),
            pl.BlockSpec((ROW_BLK, 2), lambda p, i: (p * NBLK + i, 0)),
            pl.BlockSpec((8, DIM), lambda p, i: (0, 0)),
        ],
        out_shape=[
            jax.ShapeDtypeStruct((2 * N, PACK), jnp.float32),
            jax.ShapeDtypeStruct((2 * N, 2), jnp.float32),
            jax.ShapeDtypeStruct((8, DIM), jnp.float32),
        ],
    )(h, w, asrc_mat, adst_mat)


def _agg_body(hh2, ad2, cvf, src_h, dst_h,
              num_o, den_o,
              num_s, den_s, rows, out64, adb, srcb, dstb, dstg,
              exb0, exb1, id0, id1, cv,
              sem_g, sem_a, sem_s, sem_d):
    c = lax.axis_index("c")
    s = lax.axis_index("s")

    pltpu.sync_copy(cvf, cv)

    zf16 = jnp.zeros((16,), jnp.float32)

    def _zrow(i, _):
        for q in range(HALF // 16):
            out64[i, pl.ds(q * 16, 16)] = zf16
        return 0

    lax.fori_loop(0, CHUNK, _zrow, 0)
    for g in range(CHUNK // 16):
        exb0[pl.ds(g * 16, 16)] = zf16

    rbase = s * ROWS_T

    def _zacc(j, _):
        pltpu.sync_copy(out64.at[pl.ds(0, 64)],
                        num_s.at[pl.ds(rbase + j * 64, 64)])
        return 0

    lax.fori_loop(0, ROWS_T // 64, _zacc, 0)

    def _zden(j, _):
        pltpu.sync_copy(exb0, den_s.at[pl.ds(2 * rbase + j * CHUNK, CHUNK)])
        return 0

    lax.fori_loop(0, 2 * ROWS_T // CHUNK, _zden, 0)
    plsc.subcore_barrier()

    iota16 = lax.iota(jnp.int32, 16)
    c0 = plsc.load_gather(cv, [c * 32 + iota16])
    c1 = plsc.load_gather(cv, [c * 32 + 16 + iota16])
    cn = c * N
    e_base = s * EDG_T

    def _chunk(ci, _):
        off = e_base + ci * CHUNK
        pltpu.sync_copy(src_h.at[pl.ds(off, CHUNK)], srcb)
        pltpu.sync_copy(dst_h.at[pl.ds(off, CHUNK)], dstb)

        for g in range(CHUNK // 16):
            sl = pl.ds(g * 16, 16)
            srcb[sl] = srcb[sl] + cn
            dstg[sl] = dstb[sl] + cn

        cp_r = pltpu.async_copy(hh2.at[srcb], rows, sem_g)
        cp_a = pltpu.async_copy(ad2.at[dstg], adb, sem_a)
        cp_r.wait()
        cp_a.wait()

        for g in range(CHUNK // 16):
            d16 = dstb[pl.ds(g * 16, 16)]
            r16 = g * 16 + iota16
            col0 = jnp.zeros((16,), jnp.int32)
            as0 = plsc.load_gather(rows, [r16, col0 + HALF])
            as1 = plsc.load_gather(rows, [r16, col0 + (HALF + 1)])
            ad0 = plsc.load_gather(adb, [r16, col0])
            ad1 = plsc.load_gather(adb, [r16, col0 + 1])
            p0 = as0 + ad0
            p1 = as1 + ad1
            a0 = jnp.where(p0 >= 0, p0, 0.2 * p0)
            a1 = jnp.where(p1 >= 0, p1, 0.2 * p1)
            e0 = jnp.exp(a0 - c0)
            e1 = jnp.exp(a1 - c1)
            exb0[pl.ds(g * 16, 16)] = e0
            exb1[pl.ds(g * 16, 16)] = e1
            id0[pl.ds(g * 16, 16)] = d16 * 2
            id1[pl.ds(g * 16, 16)] = d16 * 2 + 1
            for r in range(16):
                ridx = jnp.full((16,), r, jnp.int32)
                b0 = e0.at[ridx].get(mode="promise_in_bounds")
                b1 = e1.at[ridx].get(mode="promise_in_bounds")
                row = g * 16 + r
                for q in range(2):
                    out64[row, pl.ds(q * 16, 16)] = rows[row, pl.ds(q * 16, 16)] * b0
                for q in range(2, 4):
                    out64[row, pl.ds(q * 16, 16)] = rows[row, pl.ds(q * 16, 16)] * b1

        cp_n = pltpu.async_copy(out64, num_s.at[dstb], sem_s, add=True)
        cp_0 = pltpu.async_copy(exb0, den_s.at[id0], sem_d, add=True)
        cp_1 = pltpu.async_copy(exb1, den_s.at[id1], sem_d, add=True)
        cp_n.wait()
        cp_0.wait()
        cp_1.wait()
        return 0

    lax.fori_loop(0, NCHUNK, _chunk, 0)
    plsc.subcore_barrier()

    pltpu.sync_copy(num_s.at[pl.ds(rbase, ROWS_T)],
                    num_o.at[pl.ds(cn + rbase, ROWS_T)])
    pltpu.sync_copy(den_s.at[pl.ds(2 * rbase, 2 * ROWS_T)],
                    den_o.at[pl.ds(2 * cn + 2 * rbase, 2 * ROWS_T)])


def _sc_aggregate(hh2, ad2, cvf, src, dst):
    kern = pl.kernel(
        _agg_body,
        out_type=[
            jax.ShapeDtypeStruct((2 * N, HALF), jnp.float32),
            jax.ShapeDtypeStruct((4 * N,), jnp.float32),
        ],
        mesh=plsc.VectorSubcoreMesh(core_axis_name="c", subcore_axis_name="s"),
        compiler_params=pltpu.CompilerParams(needs_layout_passes=False, use_tc_tiling_on_sc=False),
        scratch_types=[
            pltpu.VMEM_SHARED((N, HALF), jnp.float32),
            pltpu.VMEM_SHARED((2 * N,), jnp.float32),
            pltpu.VMEM((CHUNK, PACK), jnp.float32),
            pltpu.VMEM((CHUNK, HALF), jnp.float32),
            pltpu.VMEM((CHUNK, 2), jnp.float32),
            pltpu.VMEM((CHUNK,), jnp.int32),
            pltpu.VMEM((CHUNK,), jnp.int32),
            pltpu.VMEM((CHUNK,), jnp.int32),
            pltpu.VMEM((CHUNK,), jnp.float32),
            pltpu.VMEM((CHUNK,), jnp.float32),
            pltpu.VMEM((CHUNK,), jnp.int32),
            pltpu.VMEM((CHUNK,), jnp.int32),
            pltpu.VMEM((64,), jnp.float32),
            pltpu.SemaphoreType.DMA,
            pltpu.SemaphoreType.DMA,
            pltpu.SemaphoreType.DMA,
            pltpu.SemaphoreType.DMA,
        ],
    )
    return kern(hh2, ad2, cvf, src, dst)


def _post_body(h_ref, nA_ref, nB_ref, dA_ref, dB_ref, b_ref, lw_ref, lb_ref,
               o_ref):
    numer = jnp.concatenate([nA_ref[...], nB_ref[...]], axis=1)
    den4 = jnp.concatenate([dA_ref[...], dB_ref[...]], axis=1)
    dv = jnp.broadcast_to(den4[:, :, None], (ROW_BLK, HEADS, HID))
    dv = dv.reshape(ROW_BLK, DIM)
    out = numer / (dv + 1e-16) + b_ref[...]
    hn = jnp.where(out > 0, out, jnp.exp(out) - 1.0)
    hsum = h_ref[...] + hn
    mu = jnp.mean(hsum, axis=-1, keepdims=True)
    var = jnp.mean((hsum - mu) ** 2, axis=-1, keepdims=True)
    o_ref[...] = (hsum - mu) / jnp.sqrt(var + 1e-5) * lw_ref[...] + lb_ref[...]


def _post(h, num, den2, b, lw, lb):
    return pl.pallas_call(
        _post_body,
        grid=(NBLK,),
        in_specs=[
            pl.BlockSpec((ROW_BLK, DIM), lambda i: (i, 0)),
            pl.BlockSpec((ROW_BLK, HALF), lambda i: (i, 0)),
            pl.BlockSpec((ROW_BLK, HALF), lambda i: (i + NBLK, 0)),
            pl.BlockSpec((ROW_BLK, 2), lambda i: (i, 0)),
            pl.BlockSpec((ROW_BLK, 2), lambda i: (i + NBLK, 0)),
            pl.BlockSpec((1, DIM), lambda i: (0, 0)),
            pl.BlockSpec((1, DIM), lambda i: (0, 0)),
            pl.BlockSpec((1, DIM), lambda i: (0, 0)),
        ],
        out_specs=pl.BlockSpec((ROW_BLK, DIM), lambda i: (i, 0)),
        out_shape=jax.ShapeDtypeStruct((N, DIM), jnp.float32),
    )(h, num, num, den2, den2, b.reshape(1, DIM), lw.reshape(1, DIM),
      lb.reshape(1, DIM))


def _pool_body(h_ref, o_ref):
    o_ref[...] = jnp.mean(h_ref[...], axis=1)


def _pool(h3):
    return pl.pallas_call(
        _pool_body,
        grid=(B // 8,),
        in_specs=[pl.BlockSpec((8, NPG, DIM), lambda i: (i, 0, 0))],
        out_specs=pl.BlockSpec((8, DIM), lambda i: (i, 0)),
        out_shape=jax.ShapeDtypeStruct((B, DIM), jnp.float32),
    )(h3)


def _vgather_body(h_hbm, idx_hbm, out_hbm, idxv, rowsv, sem):
    wid = lax.axis_index("s") * 2 + lax.axis_index("c")
    base = wid * 32
    pltpu.sync_copy(idx_hbm.at[pl.ds(base, 32)], idxv)
    pltpu.async_copy(h_hbm.at[idxv], rowsv, sem).wait()
    pltpu.sync_copy(rowsv, out_hbm.at[pl.ds(base, 32)])


def _vgather(h, idx):
    kern = pl.kernel(
        _vgather_body,
        out_type=jax.ShapeDtypeStruct((2 * B, DIM), jnp.float32),
        mesh=plsc.VectorSubcoreMesh(core_axis_name="c", subcore_axis_name="s"),
        compiler_params=pltpu.CompilerParams(needs_layout_passes=False, use_tc_tiling_on_sc=False),
        scratch_types=[
            pltpu.VMEM((32,), jnp.int32),
            pltpu.VMEM((32, DIM), jnp.float32),
            pltpu.SemaphoreType.DMA,
        ],
    )
    return kern(h, idx)


def _readout_body(v12_ref, g_ref, seat_ref, semb_ref,
                  w1_ref, b1_ref, w2_ref, b2_ref, w3_ref, b3_ref,
                  r1_ref, rb1_ref, r2_ref, rb2_ref,
                  win_ref, rank_ref):
    v1 = v12_ref[:B, :]
    v2 = v12_ref[B:, :]
    soh = (seat_ref[...] == lax.broadcasted_iota(jnp.int32, (B, 4), 1))
    sfeat = jnp.dot(soh.astype(jnp.float32), semb_ref[...],
                    preferred_element_type=jnp.float32)
    feat = jnp.concatenate([v1, v2, g_ref[...], sfeat], axis=-1)
    z = jnp.maximum(jnp.dot(feat, w1_ref[...],
                            preferred_element_type=jnp.float32) + b1_ref[...], 0.0)
    z = jnp.maximum(jnp.dot(z, w2_ref[...],
                            preferred_element_type=jnp.float32) + b2_ref[...], 0.0)
    win_ref[...] = jnp.dot(z, w3_ref[...],
                           preferred_element_type=jnp.float32) + b3_ref[...]
    r = jnp.maximum(jnp.dot(feat, r1_ref[...],
                            preferred_element_type=jnp.float32) + rb1_ref[...], 0.0)
    rk = jnp.dot(r, r2_ref[...], preferred_element_type=jnp.float32) + rb2_ref[...]
    rank_ref[...] = jax.nn.sigmoid(rk)


def _readout(v12, g, seat, p):
    return pl.pallas_call(
        _readout_body,
        out_shape=[
            jax.ShapeDtypeStruct((B, 1), jnp.float32),
            jax.ShapeDtypeStruct((B, 1), jnp.float32),
        ],
    )(v12, g, seat.reshape(B, 1), p['seat_emb'],
      p['mlp_W1'], p['mlp_b1'].reshape(1, -1),
      p['mlp_W2'], p['mlp_b2'].reshape(1, -1),
      p['mlp_W3'], p['mlp_b3'].reshape(1, -1),
      p['rk_W1'], p['rk_b1'].reshape(1, -1),
      p['rk_W2'], p['rk_b2'].reshape(1, -1))


def _att_mat(att):
    m = jnp.zeros((DIM, HEADS), jnp.float32)
    rows = jnp.arange(DIM)
    cols = jnp.repeat(jnp.arange(HEADS), HID)
    return m.at[rows, cols].set(att.reshape(-1))


def kernel(x, params, edge_index, batch, v1_local, v2_local, seat):
    src = edge_index[0]
    dst = edge_index[1]

    h = _inproj(x, params['in_W'], params['in_b'])

    for lp in params['gat']:
        hh2, ad2, st = _pre(
            h, lp['W'], _att_mat(lp['att_src']), _att_mat(lp['att_dst']))
        c4 = jnp.maximum(st[0, 0:4] + st[0, 4:8], 0.0)
        cvf = jnp.repeat(c4, 16)
        num, den = _sc_aggregate(hh2, ad2, cvf, src, dst)
        h = _post(h, num, den.reshape(2 * N, 2),
                  lp['b'], lp['ln_w'], lp['ln_b'])

    g = _pool(h.reshape(B, NPG, DIM))
    off = jnp.arange(B, dtype=jnp.int32) * NPG
    idx = jnp.concatenate([v1_local + off, v2_local + off])
    v12 = _vgather(h, idx)
    win, rank = _readout(v12, g, seat, params)
    return win, rank
